# baseline (speedup 1.0000x reference)
import functools

import jax
import jax.numpy as jnp
import numpy as np
from jax.experimental import pallas as pl
from jax.experimental.pallas import tpu as pltpu

_N = 50000
_H = 128
_BLK = 1024
_NBLK = 49
_NPAD = _BLK * _NBLK
_FMIN = float(np.finfo(np.float32).min)


def _dense_body(agg_ref, h_ref, m_ref, wrel_ref, wroot_ref, b_ref, pw_ref,
                hout_ref, sm_ref):
    agg = agg_ref[...]
    h = h_ref[...]
    m = m_ref[...]
    z = (jnp.dot(agg, wrel_ref[...], preferred_element_type=jnp.float32)
         + jnp.dot(h, wroot_ref[...], preferred_element_type=jnp.float32)
         + b_ref[...])
    z = jnp.maximum(z, 0.0) * m
    score = jnp.dot(z, pw_ref[...], preferred_element_type=jnp.float32)
    sm = jnp.where(m > 0.0, score, _FMIN)
    hout_ref[...] = z
    sm_ref[...] = sm


def _dense_layer(agg, h, m, wrel, wroot, b, pwn):
    fa = agg.shape[1]
    f = h.shape[1]
    grid = (_NBLK,)
    return pl.pallas_call(
        _dense_body,
        grid=grid,
        in_specs=[
            pl.BlockSpec((_BLK, fa), lambda i: (i, 0)),
            pl.BlockSpec((_BLK, f), lambda i: (i, 0)),
            pl.BlockSpec((_BLK, 1), lambda i: (i, 0)),
            pl.BlockSpec((fa, _H), lambda i: (0, 0)),
            pl.BlockSpec((f, _H), lambda i: (0, 0)),
            pl.BlockSpec((1, _H), lambda i: (0, 0)),
            pl.BlockSpec((_H, 1), lambda i: (0, 0)),
        ],
        out_specs=[
            pl.BlockSpec((_BLK, _H), lambda i: (i, 0)),
            pl.BlockSpec((_BLK, 1), lambda i: (i, 0)),
        ],
        out_shape=[
            jax.ShapeDtypeStruct((_NPAD, _H), jnp.float32),
            jax.ShapeDtypeStruct((_NPAD, 1), jnp.float32),
        ],
    )(agg, h, m, wrel, wroot, b, pwn)


def _pool_readout(h_new, sm, k):
    smv = sm[:, 0]
    _, idx = jax.lax.top_k(smv, k)
    new_mask = jnp.zeros((_NPAD,), dtype=bool).at[idx].set(True)
    hp = jnp.where(new_mask[:, None], h_new * jnp.tanh(smv)[:, None], 0.0)
    gmp = jnp.max(jnp.where(new_mask[:, None], hp, _FMIN), axis=0, keepdims=True)
    gap = jnp.sum(hp, axis=0, keepdims=True) / k
    return hp, new_mask, jnp.concatenate([gmp, gap], axis=1)


def kernel(x, edge_index, batch, Wrel1, Wroot1, b1, pw1, Wrel2, Wroot2, b2, pw2,
           Wrel3, Wroot3, b3, pw3, Wl1, bl1, Wl2, bl2, Wl3, bl3):
    src, dst = edge_index[0], edge_index[1]
    xpad = jnp.pad(x, ((0, _NPAD - _N), (0, 0)))
    m = jnp.pad(jnp.ones((_N, 1), jnp.float32), ((0, _NPAD - _N), (0, 0)))

    def seg(h):
        return jax.ops.segment_sum(h[src], dst, num_segments=_NPAD)

    K1, K2, K3 = 40000, 32000, 25600

    agg = seg(xpad)
    h, sm = _dense_layer(agg, xpad, m, Wrel1, Wroot1, b1.reshape(1, _H),
                         (pw1 / jnp.linalg.norm(pw1)).reshape(_H, 1))
    h, mask, x1 = _pool_readout(h, sm, K1)
    agg = seg(h)
    h, sm = _dense_layer(agg, h, mask[:, None].astype(jnp.float32), Wrel2,
                         Wroot2, b2.reshape(1, _H),
                         (pw2 / jnp.linalg.norm(pw2)).reshape(_H, 1))
    h, mask, x2 = _pool_readout(h, sm, K2)
    agg = seg(h)
    h, sm = _dense_layer(agg, h, mask[:, None].astype(jnp.float32), Wrel3,
                         Wroot3, b3.reshape(1, _H),
                         (pw3 / jnp.linalg.norm(pw3)).reshape(_H, 1))
    h, mask, x3 = _pool_readout(h, sm, K3)

    z = x1 + x2 + x3
    z = jax.nn.relu(z @ Wl1 + bl1)
    z = jax.nn.relu(z @ Wl2 + bl2)
    z = z @ Wl3 + bl3
    return z
